# baseline (speedup 1.0000x reference)
import functools
import math

import jax
import jax.numpy as jnp
from jax import lax
from jax.experimental import pallas as pl
from jax.experimental.pallas import tpu as pltpu
from jax.experimental.pallas import tpu_sc as plsc

VOCAB = 4096
HIDDEN = 4096
PAD = 1
BSZ = 2
SEQ = 8192
NPOS = PAD + 1 + SEQ

NC = 2
NS = 16
TOKENS = BSZ * SEQ
TOK_PER_W = SEQ // NS
K = 8
CHUNKS_PER_W = TOK_PER_W // K
LANES = 16
VREGS_PER_ROW = HIDDEN // LANES
UNROLL = 8


def _sinusoidal_table():
    half = HIDDEN // 2
    freq = jnp.exp(
        jnp.arange(half, dtype=jnp.float32) * (-math.log(10000.0) / (half - 1))
    )
    angles = jnp.arange(NPOS, dtype=jnp.float32)[:, None] * freq[None, :]
    table = jnp.concatenate([jnp.sin(angles), jnp.cos(angles)], axis=1)
    return table.at[PAD].set(0.0)


_mesh = plsc.VectorSubcoreMesh(core_axis_name="c", subcore_axis_name="s")


@functools.partial(
    pl.kernel,
    mesh=_mesh,
    out_type=jax.ShapeDtypeStruct((TOKENS, HIDDEN), jnp.float32),
    scratch_types=[
        pltpu.VMEM((CHUNKS_PER_W, K), jnp.int32),
        pltpu.VMEM((CHUNKS_PER_W, K), jnp.int32),
        pltpu.VMEM((K, HIDDEN), jnp.float32),
        pltpu.VMEM((K, HIDDEN), jnp.float32),
        pltpu.SemaphoreType.DMA,
        pltpu.SemaphoreType.DMA,
    ],
)
def _embed_sc(word_hbm, table_hbm, widx_hbm, pidx_hbm, out_hbm,
              widx_v, pidx_v, wbuf, tbuf, semw, semt):
    c = lax.axis_index("c")
    s = lax.axis_index("s")
    tok_base = c * SEQ + s * TOK_PER_W
    chunk_base = pl.multiple_of(tok_base // K, CHUNKS_PER_W)

    pltpu.sync_copy(widx_hbm.at[pl.ds(chunk_base, CHUNKS_PER_W)], widx_v)
    pltpu.sync_copy(pidx_hbm.at[pl.ds(chunk_base, CHUNKS_PER_W)], pidx_v)

    def step(i, carry):
        cpw = pltpu.async_copy(word_hbm.at[widx_v.at[i]], wbuf, semw)
        cpt = pltpu.async_copy(table_hbm.at[pidx_v.at[i]], tbuf, semt)
        cpw.wait()
        cpt.wait()

        def row(r, c2):
            def col(j, c3):
                base = j * (LANES * UNROLL)
                for u in range(UNROLL):
                    off = base + u * LANES
                    v = tbuf[r, pl.ds(off, LANES)]
                    plsc.addupdate(wbuf.at[r, pl.ds(off, LANES)], v)
                return c3
            lax.fori_loop(0, VREGS_PER_ROW // UNROLL, col, c2)
            return c2

        lax.fori_loop(0, K, row, 0)
        pltpu.sync_copy(wbuf, out_hbm.at[pl.ds(tok_base + i * K, K)])
        return carry

    lax.fori_loop(0, CHUNKS_PER_W, step, 0)


def kernel(input_ids, word_emb):
    mask = (input_ids != PAD).astype(jnp.int32)
    positions = (jnp.cumsum(mask, axis=1) * mask + PAD).astype(jnp.int32)
    table = _sinusoidal_table()
    widx = input_ids.reshape(TOKENS // K, K).astype(jnp.int32)
    pidx = positions.reshape(TOKENS // K, K)
    out = _embed_sc(word_emb, table, widx, pidx)
    return out.reshape(BSZ, SEQ, HIDDEN)


# speedup vs baseline: 1.7181x; 1.7181x over previous
import functools
import math

import jax
import jax.numpy as jnp
from jax import lax
from jax.experimental import pallas as pl
from jax.experimental.pallas import tpu as pltpu
from jax.experimental.pallas import tpu_sc as plsc

VOCAB = 4096
HIDDEN = 4096
PAD = 1
BSZ = 2
SEQ = 8192
NPOS = PAD + 1 + SEQ

NC = 2
NS = 16
TOKENS = BSZ * SEQ
TOK_PER_W = SEQ // NS
K = 4
CHUNKS_PER_W = TOK_PER_W // K
LANES = 16
VREGS_PER_ROW = HIDDEN // LANES
UNROLL = 8
NBUF = 2


def _sinusoidal_table():
    half = HIDDEN // 2
    freq = jnp.exp(
        jnp.arange(half, dtype=jnp.float32) * (-math.log(10000.0) / (half - 1))
    )
    angles = jnp.arange(NPOS, dtype=jnp.float32)[:, None] * freq[None, :]
    table = jnp.concatenate([jnp.sin(angles), jnp.cos(angles)], axis=1)
    return table.at[PAD].set(0.0)


_mesh = plsc.VectorSubcoreMesh(core_axis_name="c", subcore_axis_name="s")


@functools.partial(
    pl.kernel,
    mesh=_mesh,
    out_type=jax.ShapeDtypeStruct((TOKENS, HIDDEN), jnp.float32),
    scratch_types=[
        pltpu.VMEM((CHUNKS_PER_W, K), jnp.int32),
        pltpu.VMEM((CHUNKS_PER_W, K), jnp.int32),
        [pltpu.VMEM((K, HIDDEN), jnp.float32) for _ in range(NBUF)],
        [pltpu.VMEM((K, HIDDEN), jnp.float32) for _ in range(NBUF)],
        [pltpu.SemaphoreType.DMA for _ in range(NBUF)],
        [pltpu.SemaphoreType.DMA for _ in range(NBUF)],
        [pltpu.SemaphoreType.DMA for _ in range(NBUF)],
    ],
)
def _embed_sc(word_hbm, table_hbm, widx_hbm, pidx_hbm, out_hbm,
              widx_v, pidx_v, wbufs, tbufs, semws, semts, semos):
    c = lax.axis_index("c")
    s = lax.axis_index("s")
    tok_base = c * SEQ + s * TOK_PER_W
    chunk_base = pl.multiple_of(tok_base // K, CHUNKS_PER_W)

    pltpu.sync_copy(widx_hbm.at[pl.ds(chunk_base, CHUNKS_PER_W)], widx_v)
    pltpu.sync_copy(pidx_hbm.at[pl.ds(chunk_base, CHUNKS_PER_W)], pidx_v)

    def start_gathers(i, b):
        pltpu.async_copy(word_hbm.at[widx_v.at[i]], wbufs[b], semws[b])
        pltpu.async_copy(table_hbm.at[pidx_v.at[i]], tbufs[b], semts[b])

    def wait_gathers(b):
        pltpu.make_async_copy(word_hbm.at[widx_v.at[0]], wbufs[b], semws[b]).wait()
        pltpu.make_async_copy(table_hbm.at[pidx_v.at[0]], tbufs[b], semts[b]).wait()

    def add_rows(b):
        wbuf, tbuf = wbufs[b], tbufs[b]
        for r in range(K):
            def col(j, c3):
                base = j * (LANES * UNROLL)
                for u in range(UNROLL):
                    off = base + u * LANES
                    plsc.addupdate(
                        wbuf.at[r, pl.ds(off, LANES)], tbuf[r, pl.ds(off, LANES)]
                    )
                return c3
            lax.fori_loop(0, VREGS_PER_ROW // UNROLL, col, 0)

    def start_out(i, b):
        pltpu.async_copy(wbufs[b], out_hbm.at[pl.ds(tok_base + i * K, K)], semos[b])

    def wait_out(b):
        pltpu.make_async_copy(
            wbufs[b], out_hbm.at[pl.ds(tok_base, K)], semos[b]
        ).wait()

    for b in range(NBUF):
        start_gathers(b, b)

    npairs = CHUNKS_PER_W // NBUF

    def pair(p, carry):
        i0 = p * NBUF
        for b in range(NBUF):
            wait_gathers(b)
            add_rows(b)
            start_out(i0 + b, b)

        @pl.when(p + 1 < npairs)
        def _():
            for b in range(NBUF):
                wait_out(b)
                start_gathers(i0 + NBUF + b, b)

        return carry

    lax.fori_loop(0, npairs, pair, 0)
    for b in range(NBUF):
        wait_out(b)


def kernel(input_ids, word_emb):
    mask = (input_ids != PAD).astype(jnp.int32)
    positions = (jnp.cumsum(mask, axis=1) * mask + PAD).astype(jnp.int32)
    table = _sinusoidal_table()
    widx = input_ids.reshape(TOKENS // K, K).astype(jnp.int32)
    pidx = positions.reshape(TOKENS // K, K)
    out = _embed_sc(word_emb, table, widx, pidx)
    return out.reshape(BSZ, SEQ, HIDDEN)


# speedup vs baseline: 1.8674x; 1.0869x over previous
import functools
import math

import jax
import jax.numpy as jnp
from jax import lax
from jax.experimental import pallas as pl
from jax.experimental.pallas import tpu as pltpu
from jax.experimental.pallas import tpu_sc as plsc

VOCAB = 4096
HIDDEN = 4096
PAD = 1
BSZ = 2
SEQ = 8192
NPOS = PAD + 1 + SEQ

NC = 2
NS = 16
TOKENS = BSZ * SEQ
TOK_PER_W = SEQ // NS
K = 2
CHUNKS_PER_W = TOK_PER_W // K
LANES = 16
VREGS_PER_ROW = HIDDEN // LANES
UNROLL = 8
NBUF = 4


def _sinusoidal_table():
    half = HIDDEN // 2
    freq = jnp.exp(
        jnp.arange(half, dtype=jnp.float32) * (-math.log(10000.0) / (half - 1))
    )
    angles = jnp.arange(NPOS, dtype=jnp.float32)[:, None] * freq[None, :]
    table = jnp.concatenate([jnp.sin(angles), jnp.cos(angles)], axis=1)
    return table.at[PAD].set(0.0)


_mesh = plsc.VectorSubcoreMesh(core_axis_name="c", subcore_axis_name="s")


@functools.partial(
    pl.kernel,
    mesh=_mesh,
    out_type=jax.ShapeDtypeStruct((TOKENS, HIDDEN), jnp.float32),
    scratch_types=[
        pltpu.VMEM((CHUNKS_PER_W, K), jnp.int32),
        pltpu.VMEM((CHUNKS_PER_W, K), jnp.int32),
        [pltpu.VMEM((K, HIDDEN), jnp.float32) for _ in range(NBUF)],
        [pltpu.VMEM((K, HIDDEN), jnp.float32) for _ in range(NBUF)],
        [pltpu.SemaphoreType.DMA for _ in range(NBUF)],
        [pltpu.SemaphoreType.DMA for _ in range(NBUF)],
        [pltpu.SemaphoreType.DMA for _ in range(NBUF)],
    ],
)
def _embed_sc(word_hbm, table_hbm, widx_hbm, pidx_hbm, out_hbm,
              widx_v, pidx_v, wbufs, tbufs, semws, semts, semos):
    c = lax.axis_index("c")
    s = lax.axis_index("s")
    tok_base = c * SEQ + s * TOK_PER_W
    chunk_base = pl.multiple_of(tok_base // K, CHUNKS_PER_W)

    pltpu.sync_copy(widx_hbm.at[pl.ds(chunk_base, CHUNKS_PER_W)], widx_v)
    pltpu.sync_copy(pidx_hbm.at[pl.ds(chunk_base, CHUNKS_PER_W)], pidx_v)

    def start_gathers(i, b):
        pltpu.async_copy(word_hbm.at[widx_v.at[i]], wbufs[b], semws[b])
        pltpu.async_copy(table_hbm.at[pidx_v.at[i]], tbufs[b], semts[b])

    def wait_gathers(b):
        pltpu.make_async_copy(word_hbm.at[widx_v.at[0]], wbufs[b], semws[b]).wait()
        pltpu.make_async_copy(table_hbm.at[pidx_v.at[0]], tbufs[b], semts[b]).wait()

    def add_rows(b):
        wbuf, tbuf = wbufs[b], tbufs[b]
        for r in range(K):
            def col(j, c3):
                base = j * (LANES * UNROLL)
                for u in range(UNROLL):
                    off = base + u * LANES
                    plsc.addupdate(
                        wbuf.at[r, pl.ds(off, LANES)], tbuf[r, pl.ds(off, LANES)]
                    )
                return c3
            lax.fori_loop(0, VREGS_PER_ROW // UNROLL, col, 0)

    def start_out(i, b):
        pltpu.async_copy(wbufs[b], out_hbm.at[pl.ds(tok_base + i * K, K)], semos[b])

    def wait_out(b):
        pltpu.make_async_copy(
            wbufs[b], out_hbm.at[pl.ds(tok_base, K)], semos[b]
        ).wait()

    def do_step(j, b, issue):
        bp = (b - 1) % NBUF
        wait_gathers(b)
        add_rows(b)
        start_out(j, b)
        if issue:
            wait_out(bp)
            start_gathers(j + NBUF - 1, bp)

    for b in range(NBUF):
        start_gathers(b, b)
    wait_gathers(0)
    add_rows(0)
    start_out(0, 0)

    ngroups = (CHUNKS_PER_W - NBUF) // NBUF

    def group(p, carry):
        j0 = 1 + p * NBUF
        for u in range(NBUF):
            do_step(j0 + u, (1 + u) % NBUF, issue=True)
        return carry

    lax.fori_loop(0, ngroups, group, 0)

    for u in range(NBUF - 1):
        j = CHUNKS_PER_W - (NBUF - 1) + u
        b = j % NBUF
        wait_gathers(b)
        add_rows(b)
        start_out(j, b)
        wait_out((b - 1) % NBUF)
    wait_out((CHUNKS_PER_W - 1) % NBUF)


def kernel(input_ids, word_emb):
    mask = (input_ids != PAD).astype(jnp.int32)
    positions = (jnp.cumsum(mask, axis=1) * mask + PAD).astype(jnp.int32)
    table = _sinusoidal_table()
    widx = input_ids.reshape(TOKENS // K, K).astype(jnp.int32)
    pidx = positions.reshape(TOKENS // K, K)
    out = _embed_sc(word_emb, table, widx, pidx)
    return out.reshape(BSZ, SEQ, HIDDEN)


# speedup vs baseline: 3.0623x; 1.6399x over previous
import functools
import math

import numpy as np

import jax
import jax.numpy as jnp
from jax import lax
from jax.experimental import pallas as pl
from jax.experimental.pallas import tpu as pltpu
from jax.experimental.pallas import tpu_sc as plsc

VOCAB = 4096
HIDDEN = 4096
PAD = 1
BSZ = 2
SEQ = 8192
NPOS = PAD + 1 + SEQ

NC = 2
NS = 16
TOKENS = BSZ * SEQ
TOK_PER_W = SEQ // NS
K = 2
CHUNKS_PER_W = TOK_PER_W // K
LANES = 16
VREGS_PER_ROW = HIDDEN // LANES
UNROLL = 8
NBUF = 4


@functools.lru_cache(maxsize=1)
def _sinusoidal_table():
    half = HIDDEN // 2
    freq = np.exp(
        np.arange(half, dtype=np.float32) * (-math.log(10000.0) / (half - 1))
    ).astype(np.float32)
    angles = np.arange(NPOS, dtype=np.float32)[:, None] * freq[None, :]
    table = np.concatenate([np.sin(angles), np.cos(angles)], axis=1).astype(np.float32)
    table[PAD] = 0.0
    return jnp.asarray(table)


_mesh = plsc.VectorSubcoreMesh(core_axis_name="c", subcore_axis_name="s")


@functools.partial(
    pl.kernel,
    mesh=_mesh,
    out_type=jax.ShapeDtypeStruct((TOKENS, HIDDEN), jnp.float32),
    scratch_types=[
        pltpu.VMEM((CHUNKS_PER_W, K), jnp.int32),
        pltpu.VMEM((CHUNKS_PER_W, K), jnp.int32),
        [pltpu.VMEM((K, HIDDEN), jnp.float32) for _ in range(NBUF)],
        [pltpu.VMEM((K, HIDDEN), jnp.float32) for _ in range(NBUF)],
        [pltpu.SemaphoreType.DMA for _ in range(NBUF)],
        [pltpu.SemaphoreType.DMA for _ in range(NBUF)],
        [pltpu.SemaphoreType.DMA for _ in range(NBUF)],
    ],
)
def _embed_sc(word_hbm, table_hbm, widx_hbm, pidx_hbm, out_hbm,
              widx_v, pidx_v, wbufs, tbufs, semws, semts, semos):
    c = lax.axis_index("c")
    s = lax.axis_index("s")
    tok_base = c * SEQ + s * TOK_PER_W
    chunk_base = pl.multiple_of(tok_base // K, CHUNKS_PER_W)

    pltpu.sync_copy(widx_hbm.at[pl.ds(chunk_base, CHUNKS_PER_W)], widx_v)
    pltpu.sync_copy(pidx_hbm.at[pl.ds(chunk_base, CHUNKS_PER_W)], pidx_v)

    def start_gathers(i, b):
        pltpu.async_copy(word_hbm.at[widx_v.at[i]], wbufs[b], semws[b])
        pltpu.async_copy(table_hbm.at[pidx_v.at[i]], tbufs[b], semts[b])

    def wait_gathers(b):
        pltpu.make_async_copy(word_hbm.at[widx_v.at[0]], wbufs[b], semws[b]).wait()
        pltpu.make_async_copy(table_hbm.at[pidx_v.at[0]], tbufs[b], semts[b]).wait()

    def add_rows(b):
        wbuf, tbuf = wbufs[b], tbufs[b]
        for r in range(K):
            def col(j, c3):
                base = j * (LANES * UNROLL)
                for u in range(UNROLL):
                    off = base + u * LANES
                    plsc.addupdate(
                        wbuf.at[r, pl.ds(off, LANES)], tbuf[r, pl.ds(off, LANES)]
                    )
                return c3
            lax.fori_loop(0, VREGS_PER_ROW // UNROLL, col, 0)

    def start_out(i, b):
        pltpu.async_copy(wbufs[b], out_hbm.at[pl.ds(tok_base + i * K, K)], semos[b])

    def wait_out(b):
        pltpu.make_async_copy(
            wbufs[b], out_hbm.at[pl.ds(tok_base, K)], semos[b]
        ).wait()

    def do_step(j, b, issue):
        bp = (b - 1) % NBUF
        wait_gathers(b)
        add_rows(b)
        start_out(j, b)
        if issue:
            wait_out(bp)
            start_gathers(j + NBUF - 1, bp)

    for b in range(NBUF):
        start_gathers(b, b)
    wait_gathers(0)
    add_rows(0)
    start_out(0, 0)

    ngroups = (CHUNKS_PER_W - NBUF) // NBUF

    def group(p, carry):
        j0 = 1 + p * NBUF
        for u in range(NBUF):
            do_step(j0 + u, (1 + u) % NBUF, issue=True)
        return carry

    lax.fori_loop(0, ngroups, group, 0)

    for u in range(NBUF - 1):
        j = CHUNKS_PER_W - (NBUF - 1) + u
        b = j % NBUF
        wait_gathers(b)
        add_rows(b)
        start_out(j, b)
        wait_out((b - 1) % NBUF)
    wait_out((CHUNKS_PER_W - 1) % NBUF)


def kernel(input_ids, word_emb):
    mask = (input_ids != PAD).astype(jnp.int32)
    positions = (jnp.cumsum(mask, axis=1) * mask + PAD).astype(jnp.int32)
    table = _sinusoidal_table()
    widx = input_ids.reshape(TOKENS // K, K).astype(jnp.int32)
    pidx = positions.reshape(TOKENS // K, K)
    out = _embed_sc(word_emb, table, widx, pidx)
    return out.reshape(BSZ, SEQ, HIDDEN)
